# Initial kernel scaffold; baseline (speedup 1.0000x reference)
import jax
import jax.numpy as jnp
from jax.experimental import pallas as pl


def kernel(token_ids, weight):
    raise NotImplementedError("write your pallas kernel here")


# speedup vs baseline: 1.1037x; 1.1037x over previous
import functools

import jax
import jax.numpy as jnp
from jax import lax
from jax.experimental import pallas as pl
from jax.experimental.pallas import tpu as pltpu
from jax.experimental.pallas import tpu_sc as plsc

NUM_EMBEDDINGS = 1000000
EMBEDDING_DIM = 32
BATCH = 16384
HIST_LEN = 50

N_IDX = BATCH * HIST_LEN
G = 128
KG = 8
NC, NS = 2, 16
NW = NC * NS
PER_W = N_IDX // NW
GROUPS_PER_W = PER_W // G
OUTER = GROUPS_PER_W // KG
CHUNK = KG * G


def _build_gather():
    mesh = plsc.VectorSubcoreMesh(core_axis_name="c", subcore_axis_name="s")

    @functools.partial(
        pl.kernel,
        mesh=mesh,
        out_type=jax.ShapeDtypeStruct((N_IDX, EMBEDDING_DIM), jnp.float32),
        scratch_types=[
            pltpu.VMEM((GROUPS_PER_W, G), jnp.int32),
            pltpu.VMEM((CHUNK, EMBEDDING_DIM), jnp.float32),
            pltpu.SemaphoreType.DMA,
        ],
        compiler_params=pltpu.CompilerParams(use_tc_tiling_on_sc=False),
    )
    def gather_kernel(idx_hbm, table_hbm, out_hbm, idx_v, rows_v, sem):
        cid = lax.axis_index("c")
        sid = lax.axis_index("s")
        wid = sid * NC + cid
        pltpu.sync_copy(idx_hbm.at[wid], idx_v)
        base = wid * PER_W

        def body(g, carry):
            copies = []
            for j in range(KG):
                copies.append(
                    pltpu.async_copy(
                        table_hbm.at[idx_v.at[g * KG + j]],
                        rows_v.at[pl.ds(j * G, G)],
                        sem,
                    )
                )
            for c in copies:
                c.wait()
            pltpu.sync_copy(
                rows_v, out_hbm.at[pl.ds(base + g * CHUNK, CHUNK)]
            )
            return carry

        lax.fori_loop(0, OUTER, body, 0)

    return gather_kernel


_gather = _build_gather()


def kernel(token_ids, weight):
    ids = token_ids.astype(jnp.int32).reshape(NW, GROUPS_PER_W, G)
    out = _gather(ids, weight)
    return out.reshape(BATCH, HIST_LEN, EMBEDDING_DIM)


# speedup vs baseline: 1.1136x; 1.0090x over previous
import functools

import jax
import jax.numpy as jnp
from jax import lax
from jax.experimental import pallas as pl
from jax.experimental.pallas import tpu as pltpu
from jax.experimental.pallas import tpu_sc as plsc

NUM_EMBEDDINGS = 1000000
EMBEDDING_DIM = 32
BATCH = 16384
HIST_LEN = 50

N_IDX = BATCH * HIST_LEN
G = 128
KG = 8
NC, NS = 2, 16
NW = NC * NS
PER_W = N_IDX // NW
GROUPS_PER_W = PER_W // G
OUTER = GROUPS_PER_W // KG
CHUNK = KG * G


def _build_gather():
    mesh = plsc.VectorSubcoreMesh(core_axis_name="c", subcore_axis_name="s")

    @functools.partial(
        pl.kernel,
        mesh=mesh,
        out_type=jax.ShapeDtypeStruct((N_IDX, EMBEDDING_DIM), jnp.float32),
        scratch_types=[
            pltpu.VMEM((GROUPS_PER_W, G), jnp.int32),
            pltpu.VMEM((2, CHUNK, EMBEDDING_DIM), jnp.float32),
            pltpu.SemaphoreType.DMA,
            pltpu.SemaphoreType.DMA,
        ],
        compiler_params=pltpu.CompilerParams(use_tc_tiling_on_sc=False),
    )
    def gather_kernel(idx_hbm, table_hbm, out_hbm, idx_v, rows_v, sem_a, sem_b):
        cid = lax.axis_index("c")
        sid = lax.axis_index("s")
        wid = sid * NC + cid
        pltpu.sync_copy(idx_hbm.at[wid], idx_v)
        base = wid * PER_W

        def fire(g, buf, sem):
            for j in range(KG):
                pltpu.async_copy(
                    table_hbm.at[idx_v.at[g * KG + j]],
                    rows_v.at[buf].at[pl.ds(j * G, G)],
                    sem,
                )

        def drain(buf, sem):
            pltpu.make_async_copy(
                table_hbm.at[pl.ds(0, CHUNK)], rows_v.at[buf], sem
            ).wait()

        def store(g, buf):
            pltpu.sync_copy(
                rows_v.at[buf], out_hbm.at[pl.ds(base + g * CHUNK, CHUNK)]
            )

        fire(0, 0, sem_a)

        def body(gp, carry):
            g = gp * 2
            fire(g + 1, 1, sem_b)
            drain(0, sem_a)
            store(g, 0)
            fire(g + 2, 0, sem_a)
            drain(1, sem_b)
            store(g + 1, 1)
            return carry

        lax.fori_loop(0, (OUTER - 1) // 2, body, 0)
        drain(0, sem_a)
        store(OUTER - 1, 0)

    return gather_kernel


_gather = _build_gather()


def kernel(token_ids, weight):
    ids = token_ids.astype(jnp.int32).reshape(NW, GROUPS_PER_W, G)
    out = _gather(ids, weight)
    return out.reshape(BATCH, HIST_LEN, EMBEDDING_DIM)


# speedup vs baseline: 1.8947x; 1.7014x over previous
import functools

import jax
import jax.numpy as jnp
from jax import lax
from jax.experimental import pallas as pl
from jax.experimental.pallas import tpu as pltpu
from jax.experimental.pallas import tpu_sc as plsc

NUM_EMBEDDINGS = 1000000
EMBEDDING_DIM = 32
BATCH = 16384
HIST_LEN = 50

G = 128
NC, NS = 2, 16
NW = NC * NS
N_GROUPS = HIST_LEN * (BATCH // G)
G_PER_W = N_GROUPS // NW
NBUF = 8
DEPTH = 4
TC_BLK = 16


def _build_gather():
    mesh = plsc.VectorSubcoreMesh(core_axis_name="c", subcore_axis_name="s")

    @functools.partial(
        pl.kernel,
        mesh=mesh,
        out_type=jax.ShapeDtypeStruct(
            (HIST_LEN, BATCH // G, G, EMBEDDING_DIM), jnp.float32
        ),
        scratch_types=[
            pltpu.VMEM((G_PER_W, G), jnp.int32),
            pltpu.VMEM((NBUF, G, EMBEDDING_DIM), jnp.float32),
        ]
        + [pltpu.SemaphoreType.DMA] * (2 * NBUF),
        compiler_params=pltpu.CompilerParams(use_tc_tiling_on_sc=False),
    )
    def gather_kernel(idx_hbm, table_hbm, x_hbm, idx_v, rows_v, *sems):
        gsem = sems[:NBUF]
        ssem = sems[NBUF:]
        cid = lax.axis_index("c")
        sid = lax.axis_index("s")
        wid = sid * NC + cid
        pltpu.sync_copy(idx_hbm.at[wid], idx_v)
        gbase = wid * G_PER_W

        def fire(g, k):
            pltpu.async_copy(
                table_hbm.at[idx_v.at[g]], rows_v.at[k], gsem[k]
            )

        def drain_gather(k):
            pltpu.make_async_copy(
                table_hbm.at[pl.ds(0, G)], rows_v.at[k], gsem[k]
            ).wait()

        def store(g, k):
            gi = gbase + g
            h = gi // (BATCH // G)
            bc = lax.rem(gi, BATCH // G)
            pltpu.async_copy(rows_v.at[k], x_hbm.at[h, bc], ssem[k])

        def drain_stores(k):
            pltpu.make_async_copy(
                rows_v.at[k], x_hbm.at[0, 0], ssem[k]
            ).wait()

        for k in range(DEPTH):
            fire(k, k)

        def body(gp, carry):
            for k in range(NBUF):
                g = gp * NBUF + k
                nf = g + DEPTH
                kb = (k + DEPTH) % NBUF

                @pl.when(nf < G_PER_W)
                def _():
                    @pl.when(nf >= NBUF)
                    def _():
                        drain_stores(kb)

                    fire(nf, kb)

                drain_gather(k)
                store(g, k)
            return carry

        lax.fori_loop(0, G_PER_W // NBUF, body, 0)
        for k in range(NBUF):
            drain_stores(k)

    return gather_kernel


_gather = _build_gather()


def _tc_transpose_block(x_ref, y_ref):
    x = x_ref[...]
    a = x.reshape(TC_BLK, G, EMBEDDING_DIM)
    t = a.swapaxes(1, 2)
    y_ref[0] = t.reshape(TC_BLK, 4, 8, G).transpose(1, 0, 2, 3)


_transpose = pl.pallas_call(
    _tc_transpose_block,
    grid=(N_GROUPS // TC_BLK,),
    in_specs=[
        pl.BlockSpec(
            (TC_BLK, EMBEDDING_DIM, G), lambda i: (i, 0, 0)
        )
    ],
    out_specs=pl.BlockSpec(
        (1, 4, TC_BLK, 8, G),
        lambda i: (i // ((BATCH // G) // TC_BLK), 0,
                   i % ((BATCH // G) // TC_BLK), 0, 0),
    ),
    out_shape=jax.ShapeDtypeStruct(
        (HIST_LEN, 4, BATCH // G, 8, G), jnp.float32
    ),
)


def kernel(token_ids, weight):
    ids3 = token_ids.astype(jnp.int32).T.reshape(NW, G_PER_W, G)
    x = _gather(ids3, weight)
    xr = x.reshape(N_GROUPS, EMBEDDING_DIM, G)
    y = _transpose(xr)
    return (
        y.transpose(2, 4, 0, 1, 3)
        .reshape(BATCH, HIST_LEN, EMBEDDING_DIM)
    )


# speedup vs baseline: 1.9533x; 1.0309x over previous
import functools

import jax
import jax.numpy as jnp
from jax import lax
from jax.experimental import pallas as pl
from jax.experimental.pallas import tpu as pltpu
from jax.experimental.pallas import tpu_sc as plsc

NUM_EMBEDDINGS = 1000000
EMBEDDING_DIM = 32
BATCH = 16384
HIST_LEN = 50

G = 128
NC, NS = 2, 16
NW = NC * NS
N_GROUPS = HIST_LEN * (BATCH // G)
NCHUNK = 5
HC = HIST_LEN // NCHUNK
CG = HC * (BATCH // G)
G_PER_W = CG // NW
NBUF = 8
DEPTH = 4
TC_BLK = 16
BPG = (BATCH // G) // TC_BLK


def _build_gather():
    mesh = plsc.VectorSubcoreMesh(core_axis_name="c", subcore_axis_name="s")

    @functools.partial(
        pl.kernel,
        mesh=mesh,
        out_type=jax.ShapeDtypeStruct(
            (HC, BATCH // G, G, EMBEDDING_DIM), jnp.float32
        ),
        scratch_types=[
            pltpu.VMEM((G_PER_W, G), jnp.int32),
            pltpu.VMEM((NBUF, G, EMBEDDING_DIM), jnp.float32),
        ]
        + [pltpu.SemaphoreType.DMA] * (2 * NBUF),
        compiler_params=pltpu.CompilerParams(use_tc_tiling_on_sc=False),
    )
    def gather_kernel(idx_hbm, table_hbm, x_hbm, idx_v, rows_v, *sems):
        gsem = sems[:NBUF]
        ssem = sems[NBUF:]
        cid = lax.axis_index("c")
        sid = lax.axis_index("s")
        wid = sid * NC + cid
        pltpu.sync_copy(idx_hbm.at[wid], idx_v)
        gbase = wid * G_PER_W

        def fire(g, k):
            pltpu.async_copy(
                table_hbm.at[idx_v.at[g]], rows_v.at[k], gsem[k]
            )

        def drain_gather(k):
            pltpu.make_async_copy(
                table_hbm.at[pl.ds(0, G)], rows_v.at[k], gsem[k]
            ).wait()

        def store(g, k):
            gi = gbase + g
            h = gi // (BATCH // G)
            bc = lax.rem(gi, BATCH // G)
            pltpu.async_copy(rows_v.at[k], x_hbm.at[h, bc], ssem[k])

        def drain_stores(k):
            pltpu.make_async_copy(
                rows_v.at[k], x_hbm.at[0, 0], ssem[k]
            ).wait()

        for k in range(DEPTH):
            fire(k, k)

        def body(gp, carry):
            for k in range(NBUF):
                g = gp * NBUF + k
                nf = g + DEPTH
                kb = (k + DEPTH) % NBUF

                @pl.when(nf < G_PER_W)
                def _():
                    @pl.when(nf >= NBUF)
                    def _():
                        drain_stores(kb)

                    fire(nf, kb)

                drain_gather(k)
                store(g, k)
            return carry

        lax.fori_loop(0, G_PER_W // NBUF, body, 0)
        for k in range(NBUF):
            drain_stores(k)

    return gather_kernel


_gather = _build_gather()


def _ids_detile_block(x_ref, y_ref):
    y_ref[...] = x_ref[...].reshape(N_GROUPS, G)


_ids_detile = pl.pallas_call(
    _ids_detile_block,
    out_shape=jax.ShapeDtypeStruct((N_GROUPS, G), jnp.int32),
)


def _tc_transpose_block(x_ref, y_ref):
    x = x_ref[...]
    a = x.reshape(TC_BLK, G, EMBEDDING_DIM)
    t = a.swapaxes(1, 2)
    y_ref[0] = t.reshape(TC_BLK, 4, 8, G).transpose(1, 0, 2, 3)


def _tc_transpose_alias_block(y_hbm, x_ref, y_ref):
    del y_hbm
    _tc_transpose_block(x_ref, y_ref)


_Y_SHAPE = (HIST_LEN, 4, BATCH // G, 8, G)


def _make_transpose(c, aliased):
    def out_map(i):
        return (c * HC + i // BPG, 0, lax.rem(i, BPG), 0, 0)

    x_spec = pl.BlockSpec((TC_BLK, EMBEDDING_DIM, G), lambda i: (i, 0, 0))
    y_spec = pl.BlockSpec((1, 4, TC_BLK, 8, G), out_map)
    out_shape = jax.ShapeDtypeStruct(_Y_SHAPE, jnp.float32)
    if not aliased:
        return pl.pallas_call(
            _tc_transpose_block,
            grid=(CG // TC_BLK,),
            in_specs=[x_spec],
            out_specs=y_spec,
            out_shape=out_shape,
        )
    return pl.pallas_call(
        _tc_transpose_alias_block,
        grid=(CG // TC_BLK,),
        in_specs=[pl.BlockSpec(memory_space=pl.ANY), x_spec],
        out_specs=y_spec,
        out_shape=out_shape,
        input_output_aliases={0: 0},
    )


_transposes = [_make_transpose(c, c > 0) for c in range(NCHUNK)]


def kernel(token_ids, weight):
    idsq = _ids_detile(token_ids.astype(jnp.int32).T)
    y = None
    for c in range(NCHUNK):
        ids_c = idsq[c * CG:(c + 1) * CG].reshape(NW, G_PER_W, G)
        x_c = _gather(ids_c, weight)
        xr_c = x_c.reshape(CG, EMBEDDING_DIM, G)
        if c == 0:
            y = _transposes[0](xr_c)
        else:
            y = _transposes[c](y, xr_c)
    return (
        y.transpose(2, 4, 0, 1, 3)
        .reshape(BATCH, HIST_LEN, EMBEDDING_DIM)
    )
